# baseline (speedup 1.0000x reference)
import jax
import jax.numpy as jnp
from jax.experimental import pallas as pl
from jax.experimental.pallas import tpu as pltpu


def _encode(p_ref, wk_ref, bk_ref, wv_ref, bv_ref, k_ref, v_ref):
    p = p_ref[...]
    k_ref[...] = (
        jnp.dot(p, wk_ref[...].T, preferred_element_type=jnp.float32) + bk_ref[...]
    )
    v_ref[...] = (
        jnp.dot(p, wv_ref[...].T, preferred_element_type=jnp.float32) + bv_ref[...]
    )


def kernel(x, labels, prototype_vectors, Wk, bk, Wv, bv):
    n, d = prototype_vectors.shape
    a = Wk.shape[0]
    k, v = pl.pallas_call(
        _encode,
        out_shape=[
            jax.ShapeDtypeStruct((n, a), jnp.float32),
            jax.ShapeDtypeStruct((n, a), jnp.float32),
        ],
        compiler_params=pltpu.CompilerParams(
            flags={"XLA_SET_SPLIT_INPUT_OUTPUT_DMAS": True}
        ),
    )(prototype_vectors, Wk, bk.reshape(1, a), Wv, bv.reshape(1, a))
    return (k, v)
